# baseline (speedup 1.0000x reference)
import functools

import jax
import jax.numpy as jnp
from jax import lax
from jax.experimental import pallas as pl
from jax.experimental.pallas import tpu as pltpu
from jax.experimental.pallas import tpu_sc as plsc

_ALPHA = 0.01
_NC = 2
_NS = 16
_NW = _NC * _NS
_ICH = 128


def _sc_gather(centers, idx3d, b_per_w, kch):
  _, d = centers.shape
  b = b_per_w * _NW

  @functools.partial(
      pl.kernel,
      out_type=jax.ShapeDtypeStruct((b, d), jnp.float32),
      mesh=plsc.VectorSubcoreMesh(core_axis_name="c", subcore_axis_name="s"),
      compiler_params=pltpu.CompilerParams(use_tc_tiling_on_sc=False),
      scratch_types=[
          pltpu.VMEM((kch, _ICH), jnp.int32),
          pltpu.VMEM((b_per_w, d), jnp.float32),
          pltpu.SemaphoreType.DMA,
      ],
  )
  def k(table_hbm, idx_hbm, out_hbm, idx_v, rows_v, sem):
    wid = lax.axis_index("s") * _NC + lax.axis_index("c")
    pltpu.sync_copy(idx_hbm.at[wid], idx_v)
    copies = [
        pltpu.async_copy(
            table_hbm.at[idx_v.at[kk]],
            rows_v.at[pl.ds(kk * _ICH, _ICH)],
            sem,
        )
        for kk in range(kch)
    ]
    for c in copies:
      c.wait()
    pltpu.sync_copy(rows_v, out_hbm.at[pl.ds(wid * b_per_w, b_per_w)])

  return k(centers, idx3d)


def _sc_scatter(table_ref, idx3d, rows, b_per_w, kch):
  _, d = rows.shape

  @functools.partial(
      pl.kernel,
      out_type=(),
      mesh=plsc.VectorSubcoreMesh(core_axis_name="c", subcore_axis_name="s"),
      compiler_params=pltpu.CompilerParams(use_tc_tiling_on_sc=False),
      scratch_types=[
          pltpu.VMEM((kch, _ICH), jnp.int32),
          pltpu.VMEM((b_per_w, d), jnp.float32),
          pltpu.SemaphoreType.DMA,
      ],
  )
  def k(idx_hbm, rows_hbm, table_hbm, idx_v, rows_v, sem):
    wid = lax.axis_index("s") * _NC + lax.axis_index("c")
    pltpu.sync_copy(idx_hbm.at[wid], idx_v)
    pltpu.sync_copy(rows_hbm.at[pl.ds(wid * b_per_w, b_per_w)], rows_v)
    copies = [
        pltpu.async_copy(
            rows_v.at[pl.ds(kk * _ICH, _ICH)],
            table_hbm.at[idx_v.at[kk]],
            sem,
        )
        for kk in range(kch)
    ]
    for c in copies:
      c.wait()

  k(idx3d, rows, table_ref)


def _tc_combine(x, bc, lab_col, lab_row, bi):
  b, d = x.shape
  ni = b // bi

  def body(x_ref, bc_ref, lcol_ref, lrow_ref, loss_ref, new_ref):
    i = pl.program_id(0)
    xi = x_ref[pl.ds(i * bi, bi), :]
    bci = bc_ref[pl.ds(i * bi, bi), :]
    di = xi - bci
    part = jnp.sum(di * di)

    @pl.when(i == 0)
    def _():
      loss_ref[0, 0] = part

    @pl.when(i != 0)
    def _():
      loss_ref[0, 0] += part

    lc = lcol_ref[...]

    def jstep(j, s):
      xj = x_ref[pl.ds(j * bi, bi), :]
      bcj = bc_ref[pl.ds(j * bi, bi), :]
      dj = xj - bcj
      lr = lrow_ref[:, pl.ds(j * bi, bi)]
      eq = jnp.where(lc == lr, 1.0, 0.0).astype(jnp.float32)
      return s + lax.dot(eq, dj, preferred_element_type=jnp.float32)

    s = lax.fori_loop(0, ni, jstep, jnp.zeros((bi, d), jnp.float32))
    new_ref[...] = bci + _ALPHA * s

  loss, newrows = pl.pallas_call(
      body,
      grid=(ni,),
      in_specs=[
          pl.BlockSpec((b, d), lambda i: (0, 0)),
          pl.BlockSpec((b, d), lambda i: (0, 0)),
          pl.BlockSpec((bi, 1), lambda i: (i, 0)),
          pl.BlockSpec((1, b), lambda i: (0, 0)),
      ],
      out_specs=[
          pl.BlockSpec((1, 1), lambda i: (0, 0), memory_space=pltpu.SMEM),
          pl.BlockSpec((bi, d), lambda i: (i, 0)),
      ],
      out_shape=[
          jax.ShapeDtypeStruct((1, 1), jnp.float32),
          jax.ShapeDtypeStruct((b, d), jnp.float32),
      ],
  )(x, bc, lab_col, lab_row)
  return loss, newrows


def _tc_copy(t, blk):
  v, d = t.shape

  def body(in_ref, out_ref):
    out_ref[...] = in_ref[...]

  return pl.pallas_call(
      body,
      grid=(pl.cdiv(v, blk),),
      in_specs=[pl.BlockSpec((blk, d), lambda i: (i, 0))],
      out_specs=pl.BlockSpec((blk, d), lambda i: (i, 0)),
      out_shape=jax.ShapeDtypeStruct((v, d), jnp.float32),
  )(t)


def kernel(x, labels, centers):
  b, d = x.shape
  labels32 = labels.astype(jnp.int32)
  b_per_w = b // _NW
  kch = b_per_w // _ICH
  idx3d = labels32.reshape(_NW, kch, _ICH)

  bc = _sc_gather(centers, idx3d, b_per_w, kch)

  lab_col = labels32.reshape(b, 1)
  lab_row = labels32.reshape(1, b)
  loss2d, newrows = _tc_combine(x, bc, lab_col, lab_row, 1024)

  out0 = _tc_copy(centers, 8192)
  ref = jax.new_ref(out0)
  _sc_scatter(ref, idx3d, newrows, b_per_w, kch)
  return loss2d[0, 0], ref[...]


# speedup vs baseline: 1.5659x; 1.5659x over previous
import functools

import jax
import jax.numpy as jnp
from jax import lax
from jax.experimental import pallas as pl
from jax.experimental.pallas import tpu as pltpu
from jax.experimental.pallas import tpu_sc as plsc

_ALPHA = 0.01
_NC = 2
_NS = 16
_NW = _NC * _NS
_ICH = 128


def _sc_gather_ref(table_ref, idx3d, b_per_w, kch, d):
  b = b_per_w * _NW

  @functools.partial(
      pl.kernel,
      out_type=jax.ShapeDtypeStruct((b, d), jnp.float32),
      mesh=plsc.VectorSubcoreMesh(core_axis_name="c", subcore_axis_name="s"),
      compiler_params=pltpu.CompilerParams(use_tc_tiling_on_sc=False),
      scratch_types=[
          pltpu.VMEM((kch, _ICH), jnp.int32),
          pltpu.VMEM((b_per_w, d), jnp.float32),
          pltpu.SemaphoreType.DMA,
      ],
  )
  def k(table_hbm, idx_hbm, out_hbm, idx_v, rows_v, sem):
    wid = lax.axis_index("s") * _NC + lax.axis_index("c")
    pltpu.sync_copy(idx_hbm.at[wid], idx_v)
    copies = [
        pltpu.async_copy(
            table_hbm.at[idx_v.at[kk]],
            rows_v.at[pl.ds(kk * _ICH, _ICH)],
            sem,
        )
        for kk in range(kch)
    ]
    for c in copies:
      c.wait()
    pltpu.sync_copy(rows_v, out_hbm.at[pl.ds(wid * b_per_w, b_per_w)])

  return k(table_ref, idx3d)


def _sc_scatter(table_ref, idx3d, rows, b_per_w, kch):
  _, d = rows.shape

  @functools.partial(
      pl.kernel,
      out_type=(),
      mesh=plsc.VectorSubcoreMesh(core_axis_name="c", subcore_axis_name="s"),
      compiler_params=pltpu.CompilerParams(use_tc_tiling_on_sc=False),
      scratch_types=[
          pltpu.VMEM((kch, _ICH), jnp.int32),
          pltpu.VMEM((b_per_w, d), jnp.float32),
          pltpu.SemaphoreType.DMA,
      ],
  )
  def k(idx_hbm, rows_hbm, table_hbm, idx_v, rows_v, sem):
    wid = lax.axis_index("s") * _NC + lax.axis_index("c")
    pltpu.sync_copy(idx_hbm.at[wid], idx_v)
    pltpu.sync_copy(rows_hbm.at[pl.ds(wid * b_per_w, b_per_w)], rows_v)
    copies = [
        pltpu.async_copy(
            rows_v.at[pl.ds(kk * _ICH, _ICH)],
            table_hbm.at[idx_v.at[kk]],
            sem,
        )
        for kk in range(kch)
    ]
    for c in copies:
      c.wait()

  k(idx3d, rows, table_ref)


def _tc_combine(x, bc, lab_col, lab_row, bi):
  b, d = x.shape
  ni = b // bi

  def body(x_ref, bc_ref, lcol_ref, lrow_ref, loss_ref, new_ref):
    i = pl.program_id(0)
    xi = x_ref[pl.ds(i * bi, bi), :]
    bci = bc_ref[pl.ds(i * bi, bi), :]
    di = xi - bci
    part = jnp.sum(di * di)

    @pl.when(i == 0)
    def _():
      loss_ref[0, 0] = part

    @pl.when(i != 0)
    def _():
      loss_ref[0, 0] += part

    lc = lcol_ref[...]

    def jstep(j, s):
      xj = x_ref[pl.ds(j * bi, bi), :]
      bcj = bc_ref[pl.ds(j * bi, bi), :]
      dj = xj - bcj
      lr = lrow_ref[:, pl.ds(j * bi, bi)]
      eq = jnp.where(lc == lr, 1.0, 0.0).astype(jnp.bfloat16)
      return s + lax.dot(eq, dj.astype(jnp.bfloat16),
                         preferred_element_type=jnp.float32)

    s = lax.fori_loop(0, ni, jstep, jnp.zeros((bi, d), jnp.float32))
    new_ref[...] = bci + _ALPHA * s

  loss, newrows = pl.pallas_call(
      body,
      grid=(ni,),
      in_specs=[
          pl.BlockSpec((b, d), lambda i: (0, 0)),
          pl.BlockSpec((b, d), lambda i: (0, 0)),
          pl.BlockSpec((bi, 1), lambda i: (i, 0)),
          pl.BlockSpec((1, b), lambda i: (0, 0)),
      ],
      out_specs=[
          pl.BlockSpec((1, 1), lambda i: (0, 0), memory_space=pltpu.SMEM),
          pl.BlockSpec((bi, d), lambda i: (i, 0)),
      ],
      out_shape=[
          jax.ShapeDtypeStruct((1, 1), jnp.float32),
          jax.ShapeDtypeStruct((b, d), jnp.float32),
      ],
  )(x, bc, lab_col, lab_row)
  return loss, newrows


def _tc_copy(t, blk):
  v, d = t.shape

  def body(in_ref, out_ref):
    out_ref[...] = in_ref[...]

  return pl.pallas_call(
      body,
      grid=(pl.cdiv(v, blk),),
      in_specs=[pl.BlockSpec((blk, d), lambda i: (i, 0))],
      out_specs=pl.BlockSpec((blk, d), lambda i: (i, 0)),
      out_shape=jax.ShapeDtypeStruct((v, d), jnp.float32),
  )(t)


def kernel(x, labels, centers):
  b, d = x.shape
  labels32 = labels.astype(jnp.int32)
  b_per_w = b // _NW
  kch = b_per_w // _ICH
  idx3d = labels32.reshape(_NW, kch, _ICH)

  ref = jax.new_ref(centers)
  bc = _sc_gather_ref(ref, idx3d, b_per_w, kch, d)

  lab_col = labels32.reshape(b, 1)
  lab_row = labels32.reshape(1, b)
  loss2d, newrows = _tc_combine(x, bc, lab_col, lab_row, 1024)

  _sc_scatter(ref, idx3d, newrows, b_per_w, kch)
  return loss2d[0, 0], ref[...]
